# Initial kernel scaffold; baseline (speedup 1.0000x reference)
import jax
import jax.numpy as jnp
from jax.experimental import pallas as pl


def kernel(hidden_states, gate_w, w1, w2, w3):
    raise NotImplementedError("write your pallas kernel here")


# speedup vs baseline: 2.1885x; 2.1885x over previous
import functools

import jax
import jax.numpy as jnp
from jax import lax
from jax.experimental import pallas as pl
from jax.experimental.pallas import tpu as pltpu
from jax.experimental.pallas import tpu_sc as plsc

N = 8192
DM = 2048
FF = 8192
E = 8

NT = 256
T = 256
G = N * 2 // T + E
A = 4
TS = G // A
FK = 512
NF = FF // FK

SC_NC = 2
SC_NS = 16
NW = SC_NC * SC_NS
TPW = N // NW
CH_S = 32
CH_G = 16


def _router_body(x_ref, gw_ref, logits_ref, e0_ref, e1_ref, r0_ref, r1_ref,
                 tw0_ref, tw1_ref, counts_ref, carry_ref):
    t = pl.program_id(0)
    x = x_ref[...]
    logits = jnp.dot(x, gw_ref[...], preferred_element_type=jnp.float32)
    logits_ref[...] = logits
    m = jnp.max(logits, axis=-1, keepdims=True)
    ex = jnp.exp(logits - m)
    sm = ex / jnp.sum(ex, axis=-1, keepdims=True)
    lane = lax.broadcasted_iota(jnp.int32, (NT, E), 1)
    v0 = jnp.max(sm, axis=-1, keepdims=True)
    a0 = jnp.min(jnp.where(sm >= v0, lane, E), axis=-1, keepdims=True)
    sm1 = jnp.where(lane == a0, -jnp.inf, sm)
    v1 = jnp.max(sm1, axis=-1, keepdims=True)
    a1 = jnp.min(jnp.where(sm1 >= v1, lane, E), axis=-1, keepdims=True)
    tot = v0 + v1
    tw0_ref[...] = v0 / tot
    tw1_ref[...] = v1 / tot
    e0_ref[...] = a0
    e1_ref[...] = a1
    oh0 = (lane == a0).astype(jnp.float32)
    oh1 = (lane == a1).astype(jnp.float32)
    ohs = oh0 + oh1
    row = lax.broadcasted_iota(jnp.int32, (NT, NT), 0)
    col = lax.broadcasted_iota(jnp.int32, (NT, NT), 1)
    ls = (col < row).astype(jnp.float32)
    s = jnp.dot(ls, ohs, preferred_element_type=jnp.float32)

    @pl.when(t == 0)
    def _():
        carry_ref[...] = jnp.zeros_like(carry_ref)

    sc = s + carry_ref[...]
    r0_ref[...] = jnp.sum(sc * oh0, axis=-1, keepdims=True).astype(jnp.int32)
    r1_ref[...] = jnp.sum(sc * oh1, axis=-1, keepdims=True).astype(jnp.int32)
    newc = carry_ref[...] + jnp.sum(ohs, axis=0, keepdims=True)
    carry_ref[...] = newc
    counts_ref[...] = jnp.broadcast_to(newc, (8, E))


def _router(x, gate_w):
    nt_grid = N // NT
    return pl.pallas_call(
        _router_body,
        grid=(nt_grid,),
        in_specs=[
            pl.BlockSpec((NT, DM), lambda t: (t, 0)),
            pl.BlockSpec((DM, E), lambda t: (0, 0)),
        ],
        out_specs=[
            pl.BlockSpec((NT, E), lambda t: (t, 0)),
            pl.BlockSpec((NT, 1), lambda t: (t, 0)),
            pl.BlockSpec((NT, 1), lambda t: (t, 0)),
            pl.BlockSpec((NT, 1), lambda t: (t, 0)),
            pl.BlockSpec((NT, 1), lambda t: (t, 0)),
            pl.BlockSpec((NT, 1), lambda t: (t, 0)),
            pl.BlockSpec((NT, 1), lambda t: (t, 0)),
            pl.BlockSpec((8, E), lambda t: (0, 0)),
        ],
        out_shape=[
            jax.ShapeDtypeStruct((N, E), jnp.float32),
            jax.ShapeDtypeStruct((N, 1), jnp.int32),
            jax.ShapeDtypeStruct((N, 1), jnp.int32),
            jax.ShapeDtypeStruct((N, 1), jnp.int32),
            jax.ShapeDtypeStruct((N, 1), jnp.int32),
            jax.ShapeDtypeStruct((N, 1), jnp.float32),
            jax.ShapeDtypeStruct((N, 1), jnp.float32),
            jax.ShapeDtypeStruct((8, E), jnp.float32),
        ],
        scratch_shapes=[pltpu.VMEM((1, E), jnp.float32)],
    )(x, gate_w)


def _finalize_body(counts_ref, e0_ref, e1_ref, r0_ref, r1_ref,
                   pos0_ref, pos1_ref, te_ref):
    offs = []
    run = 0
    for e in range(E):
        ce = counts_ref[0, e].astype(jnp.int32)
        pe = ((ce + T - 1) // T) * T
        offs.append(run)
        run = run + pe
    e0 = e0_ref[...]
    e1 = e1_ref[...]
    acc0 = r0_ref[...]
    acc1 = r1_ref[...]
    for e in range(E):
        acc0 = acc0 + jnp.where(e0 == e, offs[e], 0)
        acc1 = acc1 + jnp.where(e1 == e, offs[e], 0)
    pos0_ref[...] = acc0
    pos1_ref[...] = acc1
    tcol = lax.broadcasted_iota(jnp.int32, (8, 128), 1)
    te = jnp.full((8, 128), -1, jnp.int32)
    for e in range(E):
        te = te + jnp.where(tcol >= offs[e] // T, 1, 0)
    te_ref[...] = te


def _finalize(counts, e0, e1, r0, r1):
    rows = N // 128
    return pl.pallas_call(
        _finalize_body,
        in_specs=[
            pl.BlockSpec(memory_space=pltpu.SMEM),
            pl.BlockSpec((rows, 128), lambda: (0, 0)),
            pl.BlockSpec((rows, 128), lambda: (0, 0)),
            pl.BlockSpec((rows, 128), lambda: (0, 0)),
            pl.BlockSpec((rows, 128), lambda: (0, 0)),
        ],
        out_specs=[
            pl.BlockSpec((rows, 128), lambda: (0, 0)),
            pl.BlockSpec((rows, 128), lambda: (0, 0)),
            pl.BlockSpec((8, 128), lambda: (0, 0)),
        ],
        out_shape=[
            jax.ShapeDtypeStruct((rows, 128), jnp.int32),
            jax.ShapeDtypeStruct((rows, 128), jnp.int32),
            jax.ShapeDtypeStruct((8, 128), jnp.int32),
        ],
    )(counts, e0, e1, r0, r1)


def _scatter(x, pos0, pos1):
    mesh = plsc.VectorSubcoreMesh(core_axis_name="c", subcore_axis_name="s")
    nchunk = TPW // CH_S

    @functools.partial(
        pl.kernel,
        mesh=mesh,
        out_type=jax.ShapeDtypeStruct((G * T, DM), jnp.float32),
        scratch_types=[
            pltpu.VMEM((CH_S, DM), jnp.float32),
            pltpu.VMEM((CH_S,), jnp.int32),
            pltpu.VMEM((CH_S,), jnp.int32),
            pltpu.SemaphoreType.DMA,
        ],
    )
    def k(x_hbm, pos0_hbm, pos1_hbm, xs_hbm, rows_v, idx0_v, idx1_v, sem):
        wid = lax.axis_index("s") * SC_NC + lax.axis_index("c")

        def chunk(i, carry):
            base = wid * TPW + i * CH_S
            pltpu.sync_copy(x_hbm.at[pl.ds(base, CH_S)], rows_v)
            pltpu.sync_copy(pos0_hbm.at[pl.ds(base, CH_S)], idx0_v)
            pltpu.sync_copy(pos1_hbm.at[pl.ds(base, CH_S)], idx1_v)
            c0 = pltpu.async_copy(rows_v, xs_hbm.at[idx0_v], sem)
            c1 = pltpu.async_copy(rows_v, xs_hbm.at[idx1_v], sem)
            c0.wait()
            c1.wait()
            return carry

        lax.fori_loop(0, nchunk, chunk, 0)

    return k(x, pos0, pos1)


def _gemm_body(te_ref, xs_ref, w1_ref, w3_ref, w2_ref, out_ref, acc_ref):
    f = pl.program_id(1)
    ti = pl.program_id(2)
    sl = pl.ds(ti * T, T)
    x = xs_ref[sl, :]
    g = jnp.dot(x, w1_ref[0], preferred_element_type=jnp.float32)
    u = jnp.dot(x, w3_ref[0], preferred_element_type=jnp.float32)
    h = (g / (1.0 + jnp.exp(-g))) * u
    part = jnp.dot(h, w2_ref[0], preferred_element_type=jnp.float32)

    @pl.when(f == 0)
    def _():
        acc_ref[sl, :] = part

    @pl.when(f > 0)
    def _():
        acc_ref[sl, :] = acc_ref[sl, :] + part

    @pl.when(f == NF - 1)
    def _():
        out_ref[...] = acc_ref[sl, :]


def _gemm(te, xs, w1, w3, w2):
    grid_spec = pltpu.PrefetchScalarGridSpec(
        num_scalar_prefetch=1,
        grid=(TS, NF, A),
        in_specs=[
            pl.BlockSpec((A * T, DM), lambda ts, f, ti, te: (ts, 0)),
            pl.BlockSpec((1, DM, FK), lambda ts, f, ti, te: (te[ts * A + ti], 0, f)),
            pl.BlockSpec((1, DM, FK), lambda ts, f, ti, te: (te[ts * A + ti], 0, f)),
            pl.BlockSpec((1, FK, DM), lambda ts, f, ti, te: (te[ts * A + ti], f, 0)),
        ],
        out_specs=pl.BlockSpec(
            (T, DM),
            lambda ts, f, ti, te: (jnp.where(f == NF - 1, ts * A + ti, G), 0)),
        scratch_shapes=[pltpu.VMEM((A * T, DM), jnp.float32)],
    )
    out = pl.pallas_call(
        _gemm_body,
        grid_spec=grid_spec,
        out_shape=jax.ShapeDtypeStruct(((G + 1) * T, DM), jnp.float32),
        compiler_params=pltpu.CompilerParams(
            dimension_semantics=("arbitrary", "arbitrary", "arbitrary"),
        ),
    )(te, xs, w1, w3, w2)
    return out


def _gather2(ys, pos0, pos1):
    mesh = plsc.VectorSubcoreMesh(core_axis_name="c", subcore_axis_name="s")
    nchunk = TPW // CH_G

    @functools.partial(
        pl.kernel,
        mesh=mesh,
        out_type=(jax.ShapeDtypeStruct((N, DM), jnp.float32),
                  jax.ShapeDtypeStruct((N, DM), jnp.float32)),
        scratch_types=[
            pltpu.VMEM((CH_G, DM), jnp.float32),
            pltpu.VMEM((CH_G, DM), jnp.float32),
            pltpu.VMEM((CH_G,), jnp.int32),
            pltpu.VMEM((CH_G,), jnp.int32),
            pltpu.SemaphoreType.DMA,
        ],
    )
    def k(ys_hbm, pos0_hbm, pos1_hbm, ya_hbm, yb_hbm,
          a_v, b_v, idx0_v, idx1_v, sem):
        wid = lax.axis_index("s") * SC_NC + lax.axis_index("c")

        def chunk(i, carry):
            base = wid * TPW + i * CH_G
            pltpu.sync_copy(pos0_hbm.at[pl.ds(base, CH_G)], idx0_v)
            pltpu.sync_copy(pos1_hbm.at[pl.ds(base, CH_G)], idx1_v)
            c0 = pltpu.async_copy(ys_hbm.at[idx0_v], a_v, sem)
            c1 = pltpu.async_copy(ys_hbm.at[idx1_v], b_v, sem)
            c0.wait()
            c1.wait()
            pltpu.sync_copy(a_v, ya_hbm.at[pl.ds(base, CH_G)])
            pltpu.sync_copy(b_v, yb_hbm.at[pl.ds(base, CH_G)])
            return carry

        lax.fori_loop(0, nchunk, chunk, 0)

    return k(ys, pos0, pos1)


def _wsum_body(ya_ref, yb_ref, tw0_ref, tw1_ref, out_ref):
    out_ref[...] = ya_ref[...] * tw0_ref[...] + yb_ref[...] * tw1_ref[...]


def _wsum(ya, yb, tw0, tw1):
    wt = 512
    return pl.pallas_call(
        _wsum_body,
        grid=(N // wt,),
        in_specs=[
            pl.BlockSpec((wt, DM), lambda t: (t, 0)),
            pl.BlockSpec((wt, DM), lambda t: (t, 0)),
            pl.BlockSpec((wt, 1), lambda t: (t, 0)),
            pl.BlockSpec((wt, 1), lambda t: (t, 0)),
        ],
        out_specs=pl.BlockSpec((wt, DM), lambda t: (t, 0)),
        out_shape=jax.ShapeDtypeStruct((N, DM), jnp.float32),
    )(ya, yb, tw0, tw1)


def kernel(hidden_states, gate_w, w1, w2, w3):
    x = hidden_states.reshape(-1, DM)
    (logits, e0, e1, r0, r1, tw0, tw1, counts) = _router(x, gate_w)
    rows = N // 128
    pos0, pos1, te = _finalize(
        counts,
        e0.reshape(rows, 128), e1.reshape(rows, 128),
        r0.reshape(rows, 128), r1.reshape(rows, 128))
    pos0 = pos0.reshape(N)
    pos1 = pos1.reshape(N)
    te_flat = te[0, :G]
    xs = _scatter(x, pos0, pos1)
    ys = _gemm(te_flat, xs, w1, w3, w2)[:G * T]
    ya, yb = _gather2(ys, pos0, pos1)
    out = _wsum(ya, yb, tw0, tw1)
    return out.reshape(hidden_states.shape), logits


# speedup vs baseline: 2.2217x; 1.0152x over previous
import functools

import jax
import jax.numpy as jnp
from jax import lax
from jax.experimental import pallas as pl
from jax.experimental.pallas import tpu as pltpu
from jax.experimental.pallas import tpu_sc as plsc

N = 8192
DM = 2048
FF = 8192
E = 8

NT = 256
T = 256
G = N * 2 // T + E
A = 6
TS = G // A
FK = 512
NF = FF // FK

SC_NC = 2
SC_NS = 16
NW = SC_NC * SC_NS
TPW = N // NW
CH_S = 32
CH_G = 16


def _router_body(x_ref, gw_ref, logits_ref, e0_ref, e1_ref, r0_ref, r1_ref,
                 tw0_ref, tw1_ref, counts_ref, carry_ref):
    t = pl.program_id(0)
    x = x_ref[...]
    logits = jnp.dot(x, gw_ref[...], preferred_element_type=jnp.float32)
    logits_ref[...] = logits
    m = jnp.max(logits, axis=-1, keepdims=True)
    ex = jnp.exp(logits - m)
    sm = ex / jnp.sum(ex, axis=-1, keepdims=True)
    lane = lax.broadcasted_iota(jnp.int32, (NT, E), 1)
    v0 = jnp.max(sm, axis=-1, keepdims=True)
    a0 = jnp.min(jnp.where(sm >= v0, lane, E), axis=-1, keepdims=True)
    sm1 = jnp.where(lane == a0, -jnp.inf, sm)
    v1 = jnp.max(sm1, axis=-1, keepdims=True)
    a1 = jnp.min(jnp.where(sm1 >= v1, lane, E), axis=-1, keepdims=True)
    tot = v0 + v1
    tw0_ref[...] = v0 / tot
    tw1_ref[...] = v1 / tot
    e0_ref[...] = a0
    e1_ref[...] = a1
    oh0 = (lane == a0).astype(jnp.float32)
    oh1 = (lane == a1).astype(jnp.float32)
    ohs = oh0 + oh1
    row = lax.broadcasted_iota(jnp.int32, (NT, NT), 0)
    col = lax.broadcasted_iota(jnp.int32, (NT, NT), 1)
    ls = (col < row).astype(jnp.float32)
    s = jnp.dot(ls, ohs, preferred_element_type=jnp.float32)

    @pl.when(t == 0)
    def _():
        carry_ref[...] = jnp.zeros_like(carry_ref)

    sc = s + carry_ref[...]
    r0_ref[...] = jnp.sum(sc * oh0, axis=-1, keepdims=True).astype(jnp.int32)
    r1_ref[...] = jnp.sum(sc * oh1, axis=-1, keepdims=True).astype(jnp.int32)
    newc = carry_ref[...] + jnp.sum(ohs, axis=0, keepdims=True)
    carry_ref[...] = newc
    counts_ref[...] = jnp.broadcast_to(newc, (8, E))


def _router(x, gate_w):
    nt_grid = N // NT
    return pl.pallas_call(
        _router_body,
        grid=(nt_grid,),
        in_specs=[
            pl.BlockSpec((NT, DM), lambda t: (t, 0)),
            pl.BlockSpec((DM, E), lambda t: (0, 0)),
        ],
        out_specs=[
            pl.BlockSpec((NT, E), lambda t: (t, 0)),
            pl.BlockSpec((NT, 1), lambda t: (t, 0)),
            pl.BlockSpec((NT, 1), lambda t: (t, 0)),
            pl.BlockSpec((NT, 1), lambda t: (t, 0)),
            pl.BlockSpec((NT, 1), lambda t: (t, 0)),
            pl.BlockSpec((NT, 1), lambda t: (t, 0)),
            pl.BlockSpec((NT, 1), lambda t: (t, 0)),
            pl.BlockSpec((8, E), lambda t: (0, 0)),
        ],
        out_shape=[
            jax.ShapeDtypeStruct((N, E), jnp.float32),
            jax.ShapeDtypeStruct((N, 1), jnp.int32),
            jax.ShapeDtypeStruct((N, 1), jnp.int32),
            jax.ShapeDtypeStruct((N, 1), jnp.int32),
            jax.ShapeDtypeStruct((N, 1), jnp.int32),
            jax.ShapeDtypeStruct((N, 1), jnp.float32),
            jax.ShapeDtypeStruct((N, 1), jnp.float32),
            jax.ShapeDtypeStruct((8, E), jnp.float32),
        ],
        scratch_shapes=[pltpu.VMEM((1, E), jnp.float32)],
    )(x, gate_w)


def _finalize_body(counts_ref, e0_ref, e1_ref, r0_ref, r1_ref,
                   pos0_ref, pos1_ref, te_ref):
    offs = []
    run = 0
    for e in range(E):
        ce = counts_ref[0, e].astype(jnp.int32)
        pe = ((ce + T - 1) // T) * T
        offs.append(run)
        run = run + pe
    e0 = e0_ref[...]
    e1 = e1_ref[...]
    acc0 = r0_ref[...]
    acc1 = r1_ref[...]
    for e in range(E):
        acc0 = acc0 + jnp.where(e0 == e, offs[e], 0)
        acc1 = acc1 + jnp.where(e1 == e, offs[e], 0)
    pos0_ref[...] = acc0
    pos1_ref[...] = acc1
    tcol = lax.broadcasted_iota(jnp.int32, (8, 128), 1)
    te = jnp.full((8, 128), -1, jnp.int32)
    for e in range(E):
        te = te + jnp.where(tcol >= offs[e] // T, 1, 0)
    te_ref[...] = te


def _finalize(counts, e0, e1, r0, r1):
    rows = N // 128
    return pl.pallas_call(
        _finalize_body,
        in_specs=[
            pl.BlockSpec(memory_space=pltpu.SMEM),
            pl.BlockSpec((rows, 128), lambda: (0, 0)),
            pl.BlockSpec((rows, 128), lambda: (0, 0)),
            pl.BlockSpec((rows, 128), lambda: (0, 0)),
            pl.BlockSpec((rows, 128), lambda: (0, 0)),
        ],
        out_specs=[
            pl.BlockSpec((rows, 128), lambda: (0, 0)),
            pl.BlockSpec((rows, 128), lambda: (0, 0)),
            pl.BlockSpec((8, 128), lambda: (0, 0)),
        ],
        out_shape=[
            jax.ShapeDtypeStruct((rows, 128), jnp.int32),
            jax.ShapeDtypeStruct((rows, 128), jnp.int32),
            jax.ShapeDtypeStruct((8, 128), jnp.int32),
        ],
    )(counts, e0, e1, r0, r1)


def _scatter(x, pos0, pos1):
    mesh = plsc.VectorSubcoreMesh(core_axis_name="c", subcore_axis_name="s")
    nchunk = TPW // CH_S

    @functools.partial(
        pl.kernel,
        mesh=mesh,
        out_type=jax.ShapeDtypeStruct((G * T, DM), jnp.float32),
        scratch_types=[
            pltpu.VMEM((CH_S, DM), jnp.float32),
            pltpu.VMEM((CH_S,), jnp.int32),
            pltpu.VMEM((CH_S,), jnp.int32),
            pltpu.SemaphoreType.DMA,
        ],
    )
    def k(x_hbm, pos0_hbm, pos1_hbm, xs_hbm, rows_v, idx0_v, idx1_v, sem):
        wid = lax.axis_index("s") * SC_NC + lax.axis_index("c")

        def chunk(i, carry):
            base = wid * TPW + i * CH_S
            pltpu.sync_copy(x_hbm.at[pl.ds(base, CH_S)], rows_v)
            pltpu.sync_copy(pos0_hbm.at[pl.ds(base, CH_S)], idx0_v)
            pltpu.sync_copy(pos1_hbm.at[pl.ds(base, CH_S)], idx1_v)
            c0 = pltpu.async_copy(rows_v, xs_hbm.at[idx0_v], sem)
            c1 = pltpu.async_copy(rows_v, xs_hbm.at[idx1_v], sem)
            c0.wait()
            c1.wait()
            return carry

        lax.fori_loop(0, nchunk, chunk, 0)

    return k(x, pos0, pos1)


def _gemm_body(te_ref, xs_ref, w1_ref, w3_ref, w2_ref, out_ref, acc_ref):
    f = pl.program_id(1)
    ti = pl.program_id(2)
    sl = pl.ds(ti * T, T)
    x = xs_ref[sl, :].astype(jnp.bfloat16)
    g = jnp.dot(x, w1_ref[0], preferred_element_type=jnp.float32)
    u = jnp.dot(x, w3_ref[0], preferred_element_type=jnp.float32)
    h = (g / (1.0 + jnp.exp(-g))) * u
    part = jnp.dot(h.astype(jnp.bfloat16), w2_ref[0],
                   preferred_element_type=jnp.float32)

    @pl.when(f == 0)
    def _():
        acc_ref[sl, :] = part

    @pl.when(f > 0)
    def _():
        acc_ref[sl, :] = acc_ref[sl, :] + part

    @pl.when(f == NF - 1)
    def _():
        out_ref[...] = acc_ref[sl, :]


def _gemm(te, xs, w1, w3, w2):
    grid_spec = pltpu.PrefetchScalarGridSpec(
        num_scalar_prefetch=1,
        grid=(TS, NF, A),
        in_specs=[
            pl.BlockSpec((A * T, DM), lambda ts, f, ti, te: (ts, 0)),
            pl.BlockSpec((1, DM, FK), lambda ts, f, ti, te: (te[ts * A + ti], 0, f)),
            pl.BlockSpec((1, DM, FK), lambda ts, f, ti, te: (te[ts * A + ti], 0, f)),
            pl.BlockSpec((1, FK, DM), lambda ts, f, ti, te: (te[ts * A + ti], f, 0)),
        ],
        out_specs=pl.BlockSpec(
            (T, DM),
            lambda ts, f, ti, te: (jnp.where(f == NF - 1, ts * A + ti, G), 0)),
        scratch_shapes=[pltpu.VMEM((A * T, DM), jnp.float32)],
    )
    out = pl.pallas_call(
        _gemm_body,
        grid_spec=grid_spec,
        out_shape=jax.ShapeDtypeStruct(((G + 1) * T, DM), jnp.float32),
        compiler_params=pltpu.CompilerParams(
            dimension_semantics=("arbitrary", "arbitrary", "arbitrary"),
        ),
    )(te, xs, w1, w3, w2)
    return out


def _gather2(ys, pos0, pos1):
    mesh = plsc.VectorSubcoreMesh(core_axis_name="c", subcore_axis_name="s")
    nchunk = TPW // CH_G

    @functools.partial(
        pl.kernel,
        mesh=mesh,
        out_type=(jax.ShapeDtypeStruct((N, DM), jnp.float32),
                  jax.ShapeDtypeStruct((N, DM), jnp.float32)),
        scratch_types=[
            pltpu.VMEM((CH_G, DM), jnp.float32),
            pltpu.VMEM((CH_G, DM), jnp.float32),
            pltpu.VMEM((CH_G,), jnp.int32),
            pltpu.VMEM((CH_G,), jnp.int32),
            pltpu.SemaphoreType.DMA,
        ],
    )
    def k(ys_hbm, pos0_hbm, pos1_hbm, ya_hbm, yb_hbm,
          a_v, b_v, idx0_v, idx1_v, sem):
        wid = lax.axis_index("s") * SC_NC + lax.axis_index("c")

        def chunk(i, carry):
            base = wid * TPW + i * CH_G
            pltpu.sync_copy(pos0_hbm.at[pl.ds(base, CH_G)], idx0_v)
            pltpu.sync_copy(pos1_hbm.at[pl.ds(base, CH_G)], idx1_v)
            c0 = pltpu.async_copy(ys_hbm.at[idx0_v], a_v, sem)
            c1 = pltpu.async_copy(ys_hbm.at[idx1_v], b_v, sem)
            c0.wait()
            c1.wait()
            pltpu.sync_copy(a_v, ya_hbm.at[pl.ds(base, CH_G)])
            pltpu.sync_copy(b_v, yb_hbm.at[pl.ds(base, CH_G)])
            return carry

        lax.fori_loop(0, nchunk, chunk, 0)

    return k(ys, pos0, pos1)


def _wsum_body(ya_ref, yb_ref, tw0_ref, tw1_ref, out_ref):
    out_ref[...] = ya_ref[...] * tw0_ref[...] + yb_ref[...] * tw1_ref[...]


def _wsum(ya, yb, tw0, tw1):
    wt = 512
    return pl.pallas_call(
        _wsum_body,
        grid=(N // wt,),
        in_specs=[
            pl.BlockSpec((wt, DM), lambda t: (t, 0)),
            pl.BlockSpec((wt, DM), lambda t: (t, 0)),
            pl.BlockSpec((wt, 1), lambda t: (t, 0)),
            pl.BlockSpec((wt, 1), lambda t: (t, 0)),
        ],
        out_specs=pl.BlockSpec((wt, DM), lambda t: (t, 0)),
        out_shape=jax.ShapeDtypeStruct((N, DM), jnp.float32),
    )(ya, yb, tw0, tw1)


def kernel(hidden_states, gate_w, w1, w2, w3):
    x = hidden_states.reshape(-1, DM)
    (logits, e0, e1, r0, r1, tw0, tw1, counts) = _router(x, gate_w)
    rows = N // 128
    pos0, pos1, te = _finalize(
        counts,
        e0.reshape(rows, 128), e1.reshape(rows, 128),
        r0.reshape(rows, 128), r1.reshape(rows, 128))
    pos0 = pos0.reshape(N)
    pos1 = pos1.reshape(N)
    te_flat = te[0, :G]
    xs = _scatter(x, pos0, pos1)
    ys = _gemm(te_flat, xs, w1.astype(jnp.bfloat16), w3.astype(jnp.bfloat16),
               w2.astype(jnp.bfloat16))[:G * T]
    ya, yb = _gather2(ys, pos0, pos1)
    out = _wsum(ya, yb, tw0, tw1)
    return out.reshape(hidden_states.shape), logits


# speedup vs baseline: 2.2726x; 1.0229x over previous
import functools

import jax
import jax.numpy as jnp
from jax import lax
from jax.experimental import pallas as pl
from jax.experimental.pallas import tpu as pltpu
from jax.experimental.pallas import tpu_sc as plsc

N = 8192
DM = 2048
FF = 8192
E = 8

NT = 256
T = 512
G = N * 2 // T + E
A = 2
TS = G // A
FK = 512
NF = FF // FK

SC_NC = 2
SC_NS = 16
NW = SC_NC * SC_NS
TPW = N // NW
CH_S = 32
CH_G = 16


def _router_body(x_ref, gw_ref, logits_ref, e0_ref, e1_ref, r0_ref, r1_ref,
                 tw0_ref, tw1_ref, counts_ref, carry_ref):
    t = pl.program_id(0)
    x = x_ref[...]
    logits = jnp.dot(x, gw_ref[...], preferred_element_type=jnp.float32)
    logits_ref[...] = logits
    m = jnp.max(logits, axis=-1, keepdims=True)
    ex = jnp.exp(logits - m)
    sm = ex / jnp.sum(ex, axis=-1, keepdims=True)
    lane = lax.broadcasted_iota(jnp.int32, (NT, E), 1)
    v0 = jnp.max(sm, axis=-1, keepdims=True)
    a0 = jnp.min(jnp.where(sm >= v0, lane, E), axis=-1, keepdims=True)
    sm1 = jnp.where(lane == a0, -jnp.inf, sm)
    v1 = jnp.max(sm1, axis=-1, keepdims=True)
    a1 = jnp.min(jnp.where(sm1 >= v1, lane, E), axis=-1, keepdims=True)
    tot = v0 + v1
    tw0_ref[...] = v0 / tot
    tw1_ref[...] = v1 / tot
    e0_ref[...] = a0
    e1_ref[...] = a1
    oh0 = (lane == a0).astype(jnp.float32)
    oh1 = (lane == a1).astype(jnp.float32)
    ohs = oh0 + oh1
    row = lax.broadcasted_iota(jnp.int32, (NT, NT), 0)
    col = lax.broadcasted_iota(jnp.int32, (NT, NT), 1)
    ls = (col < row).astype(jnp.float32)
    s = jnp.dot(ls, ohs, preferred_element_type=jnp.float32)

    @pl.when(t == 0)
    def _():
        carry_ref[...] = jnp.zeros_like(carry_ref)

    sc = s + carry_ref[...]
    r0_ref[...] = jnp.sum(sc * oh0, axis=-1, keepdims=True).astype(jnp.int32)
    r1_ref[...] = jnp.sum(sc * oh1, axis=-1, keepdims=True).astype(jnp.int32)
    newc = carry_ref[...] + jnp.sum(ohs, axis=0, keepdims=True)
    carry_ref[...] = newc
    counts_ref[...] = jnp.broadcast_to(newc, (8, E))


def _router(x, gate_w):
    nt_grid = N // NT
    return pl.pallas_call(
        _router_body,
        grid=(nt_grid,),
        in_specs=[
            pl.BlockSpec((NT, DM), lambda t: (t, 0)),
            pl.BlockSpec((DM, E), lambda t: (0, 0)),
        ],
        out_specs=[
            pl.BlockSpec((NT, E), lambda t: (t, 0)),
            pl.BlockSpec((NT, 1), lambda t: (t, 0)),
            pl.BlockSpec((NT, 1), lambda t: (t, 0)),
            pl.BlockSpec((NT, 1), lambda t: (t, 0)),
            pl.BlockSpec((NT, 1), lambda t: (t, 0)),
            pl.BlockSpec((NT, 1), lambda t: (t, 0)),
            pl.BlockSpec((NT, 1), lambda t: (t, 0)),
            pl.BlockSpec((8, E), lambda t: (0, 0)),
        ],
        out_shape=[
            jax.ShapeDtypeStruct((N, E), jnp.float32),
            jax.ShapeDtypeStruct((N, 1), jnp.int32),
            jax.ShapeDtypeStruct((N, 1), jnp.int32),
            jax.ShapeDtypeStruct((N, 1), jnp.int32),
            jax.ShapeDtypeStruct((N, 1), jnp.int32),
            jax.ShapeDtypeStruct((N, 1), jnp.float32),
            jax.ShapeDtypeStruct((N, 1), jnp.float32),
            jax.ShapeDtypeStruct((8, E), jnp.float32),
        ],
        scratch_shapes=[pltpu.VMEM((1, E), jnp.float32)],
    )(x, gate_w)


def _finalize_body(counts_ref, e0_ref, e1_ref, r0_ref, r1_ref,
                   pos0_ref, pos1_ref, te_ref):
    offs = []
    run = 0
    for e in range(E):
        ce = counts_ref[0, e].astype(jnp.int32)
        pe = ((ce + T - 1) // T) * T
        offs.append(run)
        run = run + pe
    e0 = e0_ref[...]
    e1 = e1_ref[...]
    acc0 = r0_ref[...]
    acc1 = r1_ref[...]
    for e in range(E):
        acc0 = acc0 + jnp.where(e0 == e, offs[e], 0)
        acc1 = acc1 + jnp.where(e1 == e, offs[e], 0)
    pos0_ref[...] = acc0
    pos1_ref[...] = acc1
    tcol = lax.broadcasted_iota(jnp.int32, (8, 128), 1)
    te = jnp.full((8, 128), -1, jnp.int32)
    for e in range(E):
        te = te + jnp.where(tcol >= offs[e] // T, 1, 0)
    te_ref[...] = te


def _finalize(counts, e0, e1, r0, r1):
    rows = N // 128
    return pl.pallas_call(
        _finalize_body,
        in_specs=[
            pl.BlockSpec(memory_space=pltpu.SMEM),
            pl.BlockSpec((rows, 128), lambda: (0, 0)),
            pl.BlockSpec((rows, 128), lambda: (0, 0)),
            pl.BlockSpec((rows, 128), lambda: (0, 0)),
            pl.BlockSpec((rows, 128), lambda: (0, 0)),
        ],
        out_specs=[
            pl.BlockSpec((rows, 128), lambda: (0, 0)),
            pl.BlockSpec((rows, 128), lambda: (0, 0)),
            pl.BlockSpec((8, 128), lambda: (0, 0)),
        ],
        out_shape=[
            jax.ShapeDtypeStruct((rows, 128), jnp.int32),
            jax.ShapeDtypeStruct((rows, 128), jnp.int32),
            jax.ShapeDtypeStruct((8, 128), jnp.int32),
        ],
    )(counts, e0, e1, r0, r1)


def _scatter(x, pos0, pos1):
    mesh = plsc.VectorSubcoreMesh(core_axis_name="c", subcore_axis_name="s")
    nchunk = TPW // CH_S

    @functools.partial(
        pl.kernel,
        mesh=mesh,
        out_type=jax.ShapeDtypeStruct((G * T, DM), jnp.float32),
        scratch_types=[
            pltpu.VMEM((CH_S, DM), jnp.float32),
            pltpu.VMEM((CH_S,), jnp.int32),
            pltpu.VMEM((CH_S,), jnp.int32),
            pltpu.SemaphoreType.DMA,
        ],
    )
    def k(x_hbm, pos0_hbm, pos1_hbm, xs_hbm, rows_v, idx0_v, idx1_v, sem):
        wid = lax.axis_index("s") * SC_NC + lax.axis_index("c")

        def chunk(i, carry):
            base = wid * TPW + i * CH_S
            pltpu.sync_copy(x_hbm.at[pl.ds(base, CH_S)], rows_v)
            pltpu.sync_copy(pos0_hbm.at[pl.ds(base, CH_S)], idx0_v)
            pltpu.sync_copy(pos1_hbm.at[pl.ds(base, CH_S)], idx1_v)
            c0 = pltpu.async_copy(rows_v, xs_hbm.at[idx0_v], sem)
            c1 = pltpu.async_copy(rows_v, xs_hbm.at[idx1_v], sem)
            c0.wait()
            c1.wait()
            return carry

        lax.fori_loop(0, nchunk, chunk, 0)

    return k(x, pos0, pos1)


def _gemm_body(te_ref, xs_ref, w1_ref, w3_ref, w2_ref, out_ref, acc_ref):
    f = pl.program_id(1)
    ti = pl.program_id(2)
    sl = pl.ds(ti * T, T)
    x = xs_ref[sl, :].astype(jnp.bfloat16)
    g = jnp.dot(x, w1_ref[0], preferred_element_type=jnp.float32)
    u = jnp.dot(x, w3_ref[0], preferred_element_type=jnp.float32)
    h = (g / (1.0 + jnp.exp(-g))) * u
    part = jnp.dot(h.astype(jnp.bfloat16), w2_ref[0],
                   preferred_element_type=jnp.float32)

    @pl.when(f == 0)
    def _():
        acc_ref[sl, :] = part

    @pl.when(f > 0)
    def _():
        acc_ref[sl, :] = acc_ref[sl, :] + part

    @pl.when(f == NF - 1)
    def _():
        out_ref[...] = acc_ref[sl, :]


def _gemm(te, xs, w1, w3, w2):
    grid_spec = pltpu.PrefetchScalarGridSpec(
        num_scalar_prefetch=1,
        grid=(TS, NF, A),
        in_specs=[
            pl.BlockSpec((A * T, DM), lambda ts, f, ti, te: (ts, 0)),
            pl.BlockSpec((1, DM, FK), lambda ts, f, ti, te: (te[ts * A + ti], 0, f)),
            pl.BlockSpec((1, DM, FK), lambda ts, f, ti, te: (te[ts * A + ti], 0, f)),
            pl.BlockSpec((1, FK, DM), lambda ts, f, ti, te: (te[ts * A + ti], f, 0)),
        ],
        out_specs=pl.BlockSpec(
            (T, DM),
            lambda ts, f, ti, te: (jnp.where(f == NF - 1, ts * A + ti, G), 0)),
        scratch_shapes=[pltpu.VMEM((A * T, DM), jnp.float32)],
    )
    out = pl.pallas_call(
        _gemm_body,
        grid_spec=grid_spec,
        out_shape=jax.ShapeDtypeStruct(((G + 1) * T, DM), jnp.float32),
        compiler_params=pltpu.CompilerParams(
            dimension_semantics=("arbitrary", "arbitrary", "arbitrary"),
        ),
    )(te, xs, w1, w3, w2)
    return out


def _gather2(ys, pos0, pos1):
    mesh = plsc.VectorSubcoreMesh(core_axis_name="c", subcore_axis_name="s")
    nchunk = TPW // CH_G

    @functools.partial(
        pl.kernel,
        mesh=mesh,
        out_type=(jax.ShapeDtypeStruct((N, DM), jnp.float32),
                  jax.ShapeDtypeStruct((N, DM), jnp.float32)),
        scratch_types=[
            pltpu.VMEM((CH_G, DM), jnp.float32),
            pltpu.VMEM((CH_G, DM), jnp.float32),
            pltpu.VMEM((CH_G,), jnp.int32),
            pltpu.VMEM((CH_G,), jnp.int32),
            pltpu.SemaphoreType.DMA,
        ],
    )
    def k(ys_hbm, pos0_hbm, pos1_hbm, ya_hbm, yb_hbm,
          a_v, b_v, idx0_v, idx1_v, sem):
        wid = lax.axis_index("s") * SC_NC + lax.axis_index("c")

        def chunk(i, carry):
            base = wid * TPW + i * CH_G
            pltpu.sync_copy(pos0_hbm.at[pl.ds(base, CH_G)], idx0_v)
            pltpu.sync_copy(pos1_hbm.at[pl.ds(base, CH_G)], idx1_v)
            c0 = pltpu.async_copy(ys_hbm.at[idx0_v], a_v, sem)
            c1 = pltpu.async_copy(ys_hbm.at[idx1_v], b_v, sem)
            c0.wait()
            c1.wait()
            pltpu.sync_copy(a_v, ya_hbm.at[pl.ds(base, CH_G)])
            pltpu.sync_copy(b_v, yb_hbm.at[pl.ds(base, CH_G)])
            return carry

        lax.fori_loop(0, nchunk, chunk, 0)

    return k(ys, pos0, pos1)


def _wsum_body(ya_ref, yb_ref, tw0_ref, tw1_ref, out_ref):
    out_ref[...] = ya_ref[...] * tw0_ref[...] + yb_ref[...] * tw1_ref[...]


def _wsum(ya, yb, tw0, tw1):
    wt = 512
    return pl.pallas_call(
        _wsum_body,
        grid=(N // wt,),
        in_specs=[
            pl.BlockSpec((wt, DM), lambda t: (t, 0)),
            pl.BlockSpec((wt, DM), lambda t: (t, 0)),
            pl.BlockSpec((wt, 1), lambda t: (t, 0)),
            pl.BlockSpec((wt, 1), lambda t: (t, 0)),
        ],
        out_specs=pl.BlockSpec((wt, DM), lambda t: (t, 0)),
        out_shape=jax.ShapeDtypeStruct((N, DM), jnp.float32),
    )(ya, yb, tw0, tw1)


def kernel(hidden_states, gate_w, w1, w2, w3):
    x = hidden_states.reshape(-1, DM)
    (logits, e0, e1, r0, r1, tw0, tw1, counts) = _router(x, gate_w)
    rows = N // 128
    pos0, pos1, te = _finalize(
        counts,
        e0.reshape(rows, 128), e1.reshape(rows, 128),
        r0.reshape(rows, 128), r1.reshape(rows, 128))
    pos0 = pos0.reshape(N)
    pos1 = pos1.reshape(N)
    te_flat = te[0, :G]
    xs = _scatter(x, pos0, pos1)
    ys = _gemm(te_flat, xs, w1.astype(jnp.bfloat16), w3.astype(jnp.bfloat16),
               w2.astype(jnp.bfloat16))[:G * T]
    ya, yb = _gather2(ys, pos0, pos1)
    out = _wsum(ya, yb, tw0, tw1)
    return out.reshape(hidden_states.shape), logits


# speedup vs baseline: 2.3373x; 1.0285x over previous
import functools

import jax
import jax.numpy as jnp
from jax import lax
from jax.experimental import pallas as pl
from jax.experimental.pallas import tpu as pltpu
from jax.experimental.pallas import tpu_sc as plsc

N = 8192
DM = 2048
FF = 8192
E = 8

NT = 256
T = 512
G = N * 2 // T + E
A = 2
TS = G // A
FK = 512
NF = FF // FK

SC_NC = 2
SC_NS = 16
NW = SC_NC * SC_NS
TPW = N // NW
CH_S = 32
CH_G = 16


def _router_body(x_ref, gw_ref, logits_ref, e0_ref, e1_ref, r0_ref, r1_ref,
                 tw0_ref, tw1_ref, counts_ref, carry_ref):
    t = pl.program_id(0)
    x = x_ref[...]
    logits = jnp.dot(x, gw_ref[...], preferred_element_type=jnp.float32)
    logits_ref[...] = logits
    m = jnp.max(logits, axis=-1, keepdims=True)
    ex = jnp.exp(logits - m)
    sm = ex / jnp.sum(ex, axis=-1, keepdims=True)
    lane = lax.broadcasted_iota(jnp.int32, (NT, E), 1)
    v0 = jnp.max(sm, axis=-1, keepdims=True)
    a0 = jnp.min(jnp.where(sm >= v0, lane, E), axis=-1, keepdims=True)
    sm1 = jnp.where(lane == a0, -jnp.inf, sm)
    v1 = jnp.max(sm1, axis=-1, keepdims=True)
    a1 = jnp.min(jnp.where(sm1 >= v1, lane, E), axis=-1, keepdims=True)
    tot = v0 + v1
    tw0_ref[...] = jnp.broadcast_to(v0 / tot, (NT, 16))
    tw1_ref[...] = jnp.broadcast_to(v1 / tot, (NT, 16))
    e0_ref[...] = a0
    e1_ref[...] = a1
    oh0 = (lane == a0).astype(jnp.float32)
    oh1 = (lane == a1).astype(jnp.float32)
    ohs = oh0 + oh1
    row = lax.broadcasted_iota(jnp.int32, (NT, NT), 0)
    col = lax.broadcasted_iota(jnp.int32, (NT, NT), 1)
    ls = (col < row).astype(jnp.float32)
    s = jnp.dot(ls, ohs, preferred_element_type=jnp.float32)

    @pl.when(t == 0)
    def _():
        carry_ref[...] = jnp.zeros_like(carry_ref)

    sc = s + carry_ref[...]
    r0_ref[...] = jnp.sum(sc * oh0, axis=-1, keepdims=True).astype(jnp.int32)
    r1_ref[...] = jnp.sum(sc * oh1, axis=-1, keepdims=True).astype(jnp.int32)
    newc = carry_ref[...] + jnp.sum(ohs, axis=0, keepdims=True)
    carry_ref[...] = newc
    counts_ref[...] = jnp.broadcast_to(newc, (8, E))


def _router(x, gate_w):
    nt_grid = N // NT
    return pl.pallas_call(
        _router_body,
        grid=(nt_grid,),
        in_specs=[
            pl.BlockSpec((NT, DM), lambda t: (t, 0)),
            pl.BlockSpec((DM, E), lambda t: (0, 0)),
        ],
        out_specs=[
            pl.BlockSpec((NT, E), lambda t: (t, 0)),
            pl.BlockSpec((NT, 1), lambda t: (t, 0)),
            pl.BlockSpec((NT, 1), lambda t: (t, 0)),
            pl.BlockSpec((NT, 1), lambda t: (t, 0)),
            pl.BlockSpec((NT, 1), lambda t: (t, 0)),
            pl.BlockSpec((NT, 16), lambda t: (t, 0)),
            pl.BlockSpec((NT, 16), lambda t: (t, 0)),
            pl.BlockSpec((8, E), lambda t: (0, 0)),
        ],
        out_shape=[
            jax.ShapeDtypeStruct((N, E), jnp.float32),
            jax.ShapeDtypeStruct((N, 1), jnp.int32),
            jax.ShapeDtypeStruct((N, 1), jnp.int32),
            jax.ShapeDtypeStruct((N, 1), jnp.int32),
            jax.ShapeDtypeStruct((N, 1), jnp.int32),
            jax.ShapeDtypeStruct((N, 16), jnp.float32),
            jax.ShapeDtypeStruct((N, 16), jnp.float32),
            jax.ShapeDtypeStruct((8, E), jnp.float32),
        ],
        scratch_shapes=[pltpu.VMEM((1, E), jnp.float32)],
    )(x, gate_w)


def _finalize_body(counts_ref, e0_ref, e1_ref, r0_ref, r1_ref,
                   pos0_ref, pos1_ref, te_ref):
    offs = []
    run = 0
    for e in range(E):
        ce = counts_ref[0, e].astype(jnp.int32)
        pe = ((ce + T - 1) // T) * T
        offs.append(run)
        run = run + pe
    e0 = e0_ref[...]
    e1 = e1_ref[...]
    acc0 = r0_ref[...]
    acc1 = r1_ref[...]
    for e in range(E):
        acc0 = acc0 + jnp.where(e0 == e, offs[e], 0)
        acc1 = acc1 + jnp.where(e1 == e, offs[e], 0)
    pos0_ref[...] = acc0
    pos1_ref[...] = acc1
    tcol = lax.broadcasted_iota(jnp.int32, (8, 128), 1)
    te = jnp.full((8, 128), -1, jnp.int32)
    for e in range(E):
        te = te + jnp.where(tcol >= offs[e] // T, 1, 0)
    te_ref[...] = te


def _finalize(counts, e0, e1, r0, r1):
    rows = N // 128
    return pl.pallas_call(
        _finalize_body,
        in_specs=[
            pl.BlockSpec(memory_space=pltpu.SMEM),
            pl.BlockSpec((rows, 128), lambda: (0, 0)),
            pl.BlockSpec((rows, 128), lambda: (0, 0)),
            pl.BlockSpec((rows, 128), lambda: (0, 0)),
            pl.BlockSpec((rows, 128), lambda: (0, 0)),
        ],
        out_specs=[
            pl.BlockSpec((rows, 128), lambda: (0, 0)),
            pl.BlockSpec((rows, 128), lambda: (0, 0)),
            pl.BlockSpec((8, 128), lambda: (0, 0)),
        ],
        out_shape=[
            jax.ShapeDtypeStruct((rows, 128), jnp.int32),
            jax.ShapeDtypeStruct((rows, 128), jnp.int32),
            jax.ShapeDtypeStruct((8, 128), jnp.int32),
        ],
    )(counts, e0, e1, r0, r1)


def _scatter(x, pos0, pos1):
    mesh = plsc.VectorSubcoreMesh(core_axis_name="c", subcore_axis_name="s")
    nchunk = TPW // CH_S

    @functools.partial(
        pl.kernel,
        mesh=mesh,
        out_type=jax.ShapeDtypeStruct((G * T, DM), jnp.float32),
        scratch_types=[
            pltpu.VMEM((CH_S, DM), jnp.float32),
            pltpu.VMEM((CH_S,), jnp.int32),
            pltpu.VMEM((CH_S,), jnp.int32),
            pltpu.SemaphoreType.DMA,
        ],
    )
    def k(x_hbm, pos0_hbm, pos1_hbm, xs_hbm, rows_v, idx0_v, idx1_v, sem):
        wid = lax.axis_index("s") * SC_NC + lax.axis_index("c")

        def chunk(i, carry):
            base = wid * TPW + i * CH_S
            pltpu.sync_copy(x_hbm.at[pl.ds(base, CH_S)], rows_v)
            pltpu.sync_copy(pos0_hbm.at[pl.ds(base, CH_S)], idx0_v)
            pltpu.sync_copy(pos1_hbm.at[pl.ds(base, CH_S)], idx1_v)
            c0 = pltpu.async_copy(rows_v, xs_hbm.at[idx0_v], sem)
            c1 = pltpu.async_copy(rows_v, xs_hbm.at[idx1_v], sem)
            c0.wait()
            c1.wait()
            return carry

        lax.fori_loop(0, nchunk, chunk, 0)

    return k(x, pos0, pos1)


def _gemm_body(te_ref, xs_ref, w1_ref, w3_ref, w2_ref, out_ref, acc_ref):
    f = pl.program_id(1)
    ti = pl.program_id(2)
    sl = pl.ds(ti * T, T)
    x = xs_ref[sl, :].astype(jnp.bfloat16)
    g = jnp.dot(x, w1_ref[0], preferred_element_type=jnp.float32)
    u = jnp.dot(x, w3_ref[0], preferred_element_type=jnp.float32)
    h = (g / (1.0 + jnp.exp(-g))) * u
    part = jnp.dot(h.astype(jnp.bfloat16), w2_ref[0],
                   preferred_element_type=jnp.float32)

    @pl.when(f == 0)
    def _():
        acc_ref[sl, :] = part

    @pl.when(f > 0)
    def _():
        acc_ref[sl, :] = acc_ref[sl, :] + part

    @pl.when(f == NF - 1)
    def _():
        out_ref[...] = acc_ref[sl, :]


def _gemm(te, xs, w1, w3, w2):
    grid_spec = pltpu.PrefetchScalarGridSpec(
        num_scalar_prefetch=1,
        grid=(TS, NF, A),
        in_specs=[
            pl.BlockSpec((A * T, DM), lambda ts, f, ti, te: (ts, 0)),
            pl.BlockSpec((1, DM, FK), lambda ts, f, ti, te: (te[ts * A + ti], 0, f)),
            pl.BlockSpec((1, DM, FK), lambda ts, f, ti, te: (te[ts * A + ti], 0, f)),
            pl.BlockSpec((1, FK, DM), lambda ts, f, ti, te: (te[ts * A + ti], f, 0)),
        ],
        out_specs=pl.BlockSpec(
            (T, DM),
            lambda ts, f, ti, te: (jnp.where(f == NF - 1, ts * A + ti, G), 0)),
        scratch_shapes=[pltpu.VMEM((A * T, DM), jnp.float32)],
    )
    out = pl.pallas_call(
        _gemm_body,
        grid_spec=grid_spec,
        out_shape=jax.ShapeDtypeStruct(((G + 1) * T, DM), jnp.float32),
        compiler_params=pltpu.CompilerParams(
            dimension_semantics=("arbitrary", "arbitrary", "arbitrary"),
        ),
    )(te, xs, w1, w3, w2)
    return out


def _combine(ys, pos0, pos1, tw0x, tw1x):
    mesh = plsc.VectorSubcoreMesh(core_axis_name="c", subcore_axis_name="s")
    nchunk = TPW // CH_G

    @functools.partial(
        pl.kernel,
        mesh=mesh,
        out_type=jax.ShapeDtypeStruct((N, DM), jnp.float32),
        scratch_types=[
            pltpu.VMEM((CH_G, DM), jnp.float32),
            pltpu.VMEM((CH_G, DM), jnp.float32),
            pltpu.VMEM((CH_G,), jnp.int32),
            pltpu.VMEM((CH_G,), jnp.int32),
            pltpu.VMEM((CH_G, 16), jnp.float32),
            pltpu.VMEM((CH_G, 16), jnp.float32),
            pltpu.SemaphoreType.DMA,
        ],
    )
    def k(ys_hbm, pos0_hbm, pos1_hbm, tw0_hbm, tw1_hbm, out_hbm,
          a_v, b_v, idx0_v, idx1_v, w0_v, w1_v, sem):
        wid = lax.axis_index("s") * SC_NC + lax.axis_index("c")

        def chunk(i, carry):
            base = wid * TPW + i * CH_G
            pltpu.sync_copy(pos0_hbm.at[pl.ds(base, CH_G)], idx0_v)
            pltpu.sync_copy(pos1_hbm.at[pl.ds(base, CH_G)], idx1_v)
            pltpu.sync_copy(tw0_hbm.at[pl.ds(base, CH_G)], w0_v)
            pltpu.sync_copy(tw1_hbm.at[pl.ds(base, CH_G)], w1_v)
            c0 = pltpu.async_copy(ys_hbm.at[idx0_v], a_v, sem)
            c1 = pltpu.async_copy(ys_hbm.at[idx1_v], b_v, sem)
            c0.wait()
            c1.wait()

            def row(r, c2):
                w0r = w0_v[r, :]
                w1r = w1_v[r, :]
                for c in range(DM // 16):
                    cs = pl.ds(c * 16, 16)
                    a_v[r, cs] = a_v[r, cs] * w0r + b_v[r, cs] * w1r
                return c2

            lax.fori_loop(0, CH_G, row, 0)
            pltpu.sync_copy(a_v, out_hbm.at[pl.ds(base, CH_G)])
            return carry

        lax.fori_loop(0, nchunk, chunk, 0)

    return k(ys, pos0, pos1, tw0x, tw1x)


def kernel(hidden_states, gate_w, w1, w2, w3):
    x = hidden_states.reshape(-1, DM)
    (logits, e0, e1, r0, r1, tw0, tw1, counts) = _router(x, gate_w)
    rows = N // 128
    pos0, pos1, te = _finalize(
        counts,
        e0.reshape(rows, 128), e1.reshape(rows, 128),
        r0.reshape(rows, 128), r1.reshape(rows, 128))
    pos0 = pos0.reshape(N)
    pos1 = pos1.reshape(N)
    te_flat = te[0, :G]
    xs = _scatter(x, pos0, pos1)
    ys = _gemm(te_flat, xs, w1.astype(jnp.bfloat16), w3.astype(jnp.bfloat16),
               w2.astype(jnp.bfloat16))
    out = _combine(ys, pos0, pos1, tw0, tw1)
    return out.reshape(hidden_states.shape), logits
